# baseline (speedup 1.0000x reference)
import jax
import jax.numpy as jnp
from jax import lax
from jax.experimental import pallas as pl
from jax.experimental.pallas import tpu as pltpu

R = 16


def _body(thresh_smem, embs_ref, tbl_ref, out_ref):
    ld = tbl_ref.shape[0]
    i = pl.program_id(0)
    tbl = tbl_ref[...]
    col = lax.iota(jnp.int32, ld)
    for j in range(R):
        sl_j = pl.ds(j * ld, ld)
        row = embs_ref[sl_j]
        thr = thresh_smem[i * R + j]
        out_ref[sl_j] = row + jnp.where(col < thr, tbl, 0.0)


def kernel(embs, seq_lengths, table):
    B, L, D = embs.shape
    LD = L * D
    CH = R * LD
    embs1 = embs.reshape(B * LD)
    tbl = table[1:L + 1].reshape(LD)
    thresh = seq_lengths.astype(jnp.int32) * D

    grid_spec = pltpu.PrefetchScalarGridSpec(
        num_scalar_prefetch=1,
        grid=(B // R,),
        in_specs=[
            pl.BlockSpec((CH,), lambda i, *_: (i,)),
            pl.BlockSpec((LD,), lambda i, *_: (0,)),
        ],
        out_specs=pl.BlockSpec((CH,), lambda i, *_: (i,)),
    )
    out = pl.pallas_call(
        _body,
        grid_spec=grid_spec,
        out_shape=jax.ShapeDtypeStruct((B * LD,), jnp.float32),
    )(thresh, embs1, tbl)
    return out.reshape(B, L, D)
